# baseline (speedup 1.0000x reference)
import functools

import jax
import jax.numpy as jnp
from jax import lax
from jax.experimental import pallas as pl
from jax.experimental.pallas import tpu as pltpu
from jax.experimental.pallas import tpu_sc as plsc

B, T, E = 8, 2048, 256
S, WMAX = 512, 10
H = 150
CHUNK = 256

_NC, _NS = 2, 16
_NW = _NC * _NS


def _attn_prefix_body(x_ref, w1_ref, b1_ref, w2_ref, b2_ref, w3_ref, b3_ref,
                      q_ref):
    x = x_ref[0]
    h = jnp.maximum(
        jnp.dot(x, w1_ref[...], preferred_element_type=jnp.float32)
        + b1_ref[...], 0.0)
    h = jnp.maximum(
        jnp.dot(h, w2_ref[...], preferred_element_type=jnp.float32)
        + b2_ref[...], 0.0)
    a = (jnp.dot(h, w3_ref[...], preferred_element_type=jnp.float32)
         + b3_ref[...])
    z = x * a

    tri = (lax.broadcasted_iota(jnp.int32, (CHUNK, CHUNK), 0)
           > lax.broadcasted_iota(jnp.int32, (CHUNK, CHUNK), 1)
           ).astype(jnp.float32)
    nchunk = T // CHUNK
    sums = [jnp.sum(z[i * CHUNK:(i + 1) * CHUNK, :], axis=0, keepdims=True)
            for i in range(nchunk)]
    carry = jnp.zeros((1, E), jnp.float32)
    carries = []
    for i in range(nchunk):
        carries.append(carry)
        carry = carry + sums[i]
    for i in range(nchunk):
        zc = z[i * CHUNK:(i + 1) * CHUNK, :]
        q_ref[0, i * CHUNK:(i + 1) * CHUNK, :] = (
            jnp.dot(tri, zc, preferred_element_type=jnp.float32) + carries[i])


def _attn_prefix(batch_embeds, w1, b1, w2, b2, w3, b3):
    return pl.pallas_call(
        _attn_prefix_body,
        grid=(B,),
        in_specs=[
            pl.BlockSpec((1, T, E), lambda b: (b, 0, 0)),
            pl.BlockSpec((E, H), lambda b: (0, 0)),
            pl.BlockSpec((1, H), lambda b: (0, 0)),
            pl.BlockSpec((H, H), lambda b: (0, 0)),
            pl.BlockSpec((1, H), lambda b: (0, 0)),
            pl.BlockSpec((H, 1), lambda b: (0, 0)),
            pl.BlockSpec((1, 1), lambda b: (0, 0)),
        ],
        out_specs=pl.BlockSpec((1, T, E), lambda b: (b, 0, 0)),
        out_shape=jax.ShapeDtypeStruct((B, T, E), jnp.float32),
    )(batch_embeds, w1, b1.reshape(1, H), w2, b2.reshape(1, H), w3,
      b3.reshape(1, 1))


_PW = (B * S) // _NW


def _sc_gather_pair(table, starts, widths, plus_one):
    d = table.shape[1]
    n = B * S
    mesh = plsc.VectorSubcoreMesh(core_axis_name="c", subcore_axis_name="s")
    row = jax.ShapeDtypeStruct((n, d), table.dtype)

    @functools.partial(
        pl.kernel,
        mesh=mesh,
        out_type=[row, row],
        scratch_types=[
            pltpu.VMEM((_PW,), jnp.int32),
            pltpu.VMEM((_PW,), jnp.int32),
            pltpu.VMEM((_PW,), jnp.int32),
            pltpu.VMEM((_PW,), jnp.int32),
            pltpu.VMEM((_PW, d), jnp.float32),
            pltpu.VMEM((_PW, d), jnp.float32),
            pltpu.SemaphoreType.DMA,
            pltpu.SemaphoreType.DMA,
        ],
    )
    def k(t_hbm, st_hbm, wd_hbm, oa_hbm, ob_hbm, st_v, wd_v, fa_v, fb_v,
          ra_v, rb_v, sa, sb):
        wid = lax.axis_index("s") * _NC + lax.axis_index("c")
        base = wid * _PW
        boff = (wid // (S // _PW)) * T
        pltpu.sync_copy(st_hbm.at[pl.ds(base, _PW)], st_v)
        pltpu.sync_copy(wd_hbm.at[pl.ds(base, _PW)], wd_v)

        @pl.loop(0, _PW, step=16)
        def _(i):
            s16 = st_v[pl.ds(i, 16)] + boff
            e16 = s16 + wd_v[pl.ds(i, 16)]
            fa_v[pl.ds(i, 16)] = s16
            fb_v[pl.ds(i, 16)] = e16 + plus_one

        ca = pltpu.async_copy(t_hbm.at[fa_v], ra_v, sa)
        cb = pltpu.async_copy(t_hbm.at[fb_v], rb_v, sb)
        ca.wait()
        pltpu.sync_copy(ra_v, oa_hbm.at[pl.ds(base, _PW)])
        cb.wait()
        pltpu.sync_copy(rb_v, ob_hbm.at[pl.ds(base, _PW)])

    return k(table, starts, widths)


_ROWS = 512


def _mention_body(gs_ref, ge_ref, qs_ref, qe_ref, w1_ref, b1_ref, w2_ref,
                  b2_ref, w3_ref, b3_ref, se_ref, ms_ref):
    gs = gs_ref[...]
    ge = ge_ref[...]
    w = qe_ref[...] - qs_ref[...]
    se = jnp.concatenate([gs, ge, w], axis=1)
    se_ref[...] = se
    h = jnp.maximum(
        jnp.dot(se, w1_ref[...], preferred_element_type=jnp.float32)
        + b1_ref[...], 0.0)
    h = jnp.maximum(
        jnp.dot(h, w2_ref[...], preferred_element_type=jnp.float32)
        + b2_ref[...], 0.0)
    ms_ref[...] = (jnp.dot(h, w3_ref[...], preferred_element_type=jnp.float32)
                   + b3_ref[...])


def _mention(gs, ge, qs, qe, w1, b1, w2, b2, w3, b3):
    n = gs.shape[0]
    row_spec = pl.BlockSpec((_ROWS, E), lambda i: (i, 0))
    return pl.pallas_call(
        _mention_body,
        grid=(n // _ROWS,),
        in_specs=[
            row_spec, row_spec, row_spec, row_spec,
            pl.BlockSpec((3 * E, H), lambda i: (0, 0)),
            pl.BlockSpec((1, H), lambda i: (0, 0)),
            pl.BlockSpec((H, H), lambda i: (0, 0)),
            pl.BlockSpec((1, H), lambda i: (0, 0)),
            pl.BlockSpec((H, 1), lambda i: (0, 0)),
            pl.BlockSpec((1, 1), lambda i: (0, 0)),
        ],
        out_specs=[
            pl.BlockSpec((_ROWS, 3 * E), lambda i: (i, 0)),
            pl.BlockSpec((_ROWS, 1), lambda i: (i, 0)),
        ],
        out_shape=[
            jax.ShapeDtypeStruct((n, 3 * E), jnp.float32),
            jax.ShapeDtypeStruct((n, 1), jnp.float32),
        ],
    )(gs, ge, qs, qe, w1, b1.reshape(1, H), w2, b2.reshape(1, H), w3,
      b3.reshape(1, 1))


def kernel(batch_embeds, span_starts, span_widths, Wa1, ba1, Wa2, ba2, Wa3,
           ba3, Ws1, bs1, Ws2, bs2, Ws3, bs3):
    starts = span_starts.astype(jnp.int32).reshape(-1)
    widths = span_widths.astype(jnp.int32).reshape(-1)

    emb_flat = batch_embeds.reshape(B * T, E)
    gs, ge = _sc_gather_pair(emb_flat, starts, widths, plus_one=0)
    q = _attn_prefix(batch_embeds, Wa1, ba1, Wa2, ba2, Wa3, ba3)
    qs, qe = _sc_gather_pair(q.reshape(B * T, E), starts, widths, plus_one=1)
    span_embeds, scores = _mention(gs, ge, qs, qe,
                                   Ws1, bs1, Ws2, bs2, Ws3, bs3)
    return span_embeds.reshape(B, S, 3 * E), scores.reshape(B, S, 1)


# speedup vs baseline: 1.0695x; 1.0695x over previous
import functools

import jax
import jax.numpy as jnp
from jax import lax
from jax.experimental import pallas as pl
from jax.experimental.pallas import tpu as pltpu
from jax.experimental.pallas import tpu_sc as plsc

B, T, E = 8, 2048, 256
S, WMAX = 512, 10
H = 150
CHUNK = 256

_NC, _NS = 2, 16
_NW = _NC * _NS


def _attn_prefix_body(x_ref, w1_ref, b1_ref, w2_ref, b2_ref, w3_ref, b3_ref,
                      q_ref):
    x = x_ref[0]
    h = jnp.maximum(
        jnp.dot(x, w1_ref[...], preferred_element_type=jnp.float32)
        + b1_ref[...], 0.0)
    h = jnp.maximum(
        jnp.dot(h, w2_ref[...], preferred_element_type=jnp.float32)
        + b2_ref[...], 0.0)
    a = (jnp.dot(h, w3_ref[...], preferred_element_type=jnp.float32)
         + b3_ref[...])
    z = x * a

    tri = (lax.broadcasted_iota(jnp.int32, (CHUNK, CHUNK), 0)
           > lax.broadcasted_iota(jnp.int32, (CHUNK, CHUNK), 1)
           ).astype(jnp.float32)
    nchunk = T // CHUNK
    sums = [jnp.sum(z[i * CHUNK:(i + 1) * CHUNK, :], axis=0, keepdims=True)
            for i in range(nchunk)]
    carry = jnp.zeros((1, E), jnp.float32)
    carries = []
    for i in range(nchunk):
        carries.append(carry)
        carry = carry + sums[i]
    for i in range(nchunk):
        zc = z[i * CHUNK:(i + 1) * CHUNK, :]
        q_ref[0, i * CHUNK:(i + 1) * CHUNK, :] = (
            jnp.dot(tri, zc, preferred_element_type=jnp.float32) + carries[i])


def _attn_prefix(batch_embeds, w1, b1, w2, b2, w3, b3):
    return pl.pallas_call(
        _attn_prefix_body,
        grid=(B,),
        in_specs=[
            pl.BlockSpec((1, T, E), lambda b: (b, 0, 0)),
            pl.BlockSpec((E, H), lambda b: (0, 0)),
            pl.BlockSpec((1, H), lambda b: (0, 0)),
            pl.BlockSpec((H, H), lambda b: (0, 0)),
            pl.BlockSpec((1, H), lambda b: (0, 0)),
            pl.BlockSpec((H, 1), lambda b: (0, 0)),
            pl.BlockSpec((1, 1), lambda b: (0, 0)),
        ],
        out_specs=pl.BlockSpec((1, T, E), lambda b: (b, 0, 0)),
        out_shape=jax.ShapeDtypeStruct((B, T, E), jnp.float32),
    )(batch_embeds, w1, b1.reshape(1, H), w2, b2.reshape(1, H), w3,
      b3.reshape(1, 1))


_PW = (B * S) // _NW


def _sc_gather4(emb_flat, q_flat, starts, widths):
    d = emb_flat.shape[1]
    n = B * S
    mesh = plsc.VectorSubcoreMesh(core_axis_name="c", subcore_axis_name="s")
    row = jax.ShapeDtypeStruct((n, d), emb_flat.dtype)

    @functools.partial(
        pl.kernel,
        mesh=mesh,
        out_type=[row, row, row, row],
        scratch_types=[
            pltpu.VMEM((_PW,), jnp.int32),
            pltpu.VMEM((_PW,), jnp.int32),
            pltpu.VMEM((_PW,), jnp.int32),
            pltpu.VMEM((_PW,), jnp.int32),
            pltpu.VMEM((_PW,), jnp.int32),
            pltpu.VMEM((_PW, d), jnp.float32),
            pltpu.VMEM((_PW, d), jnp.float32),
            pltpu.SemaphoreType.DMA,
            pltpu.SemaphoreType.DMA,
        ],
    )
    def k(emb_hbm, q_hbm, st_hbm, wd_hbm, gs_hbm, ge_hbm, qs_hbm, qe_hbm,
          st_v, wd_v, fs_v, fe_v, fq_v, ra_v, rb_v, sa, sb):
        wid = lax.axis_index("s") * _NC + lax.axis_index("c")
        base = wid * _PW
        boff = (wid // (S // _PW)) * T
        pltpu.sync_copy(st_hbm.at[pl.ds(base, _PW)], st_v)
        pltpu.sync_copy(wd_hbm.at[pl.ds(base, _PW)], wd_v)

        @pl.loop(0, _PW, step=16)
        def _(i):
            s16 = st_v[pl.ds(i, 16)] + boff
            e16 = s16 + wd_v[pl.ds(i, 16)]
            fs_v[pl.ds(i, 16)] = s16
            fe_v[pl.ds(i, 16)] = e16
            fq_v[pl.ds(i, 16)] = e16 + 1

        cp = pltpu.async_copy(emb_hbm.at[fs_v], ra_v, sa)
        cq = pltpu.async_copy(emb_hbm.at[fe_v], rb_v, sb)
        cp.wait()
        pltpu.sync_copy(ra_v, gs_hbm.at[pl.ds(base, _PW)])
        cq.wait()
        pltpu.sync_copy(rb_v, ge_hbm.at[pl.ds(base, _PW)])
        cp = pltpu.async_copy(q_hbm.at[fs_v], ra_v, sa)
        cq = pltpu.async_copy(q_hbm.at[fq_v], rb_v, sb)
        cp.wait()
        pltpu.sync_copy(ra_v, qs_hbm.at[pl.ds(base, _PW)])
        cq.wait()
        pltpu.sync_copy(rb_v, qe_hbm.at[pl.ds(base, _PW)])

    return k(emb_flat, q_flat, starts, widths)


_ROWS = 512


def _mention_body(gs_ref, ge_ref, qs_ref, qe_ref, w1_ref, b1_ref, w2_ref,
                  b2_ref, w3_ref, b3_ref, se_ref, ms_ref):
    gs = gs_ref[...]
    ge = ge_ref[...]
    w = qe_ref[...] - qs_ref[...]
    se = jnp.concatenate([gs, ge, w], axis=1)
    se_ref[...] = se
    h = jnp.maximum(
        jnp.dot(se, w1_ref[...], preferred_element_type=jnp.float32)
        + b1_ref[...], 0.0)
    h = jnp.maximum(
        jnp.dot(h, w2_ref[...], preferred_element_type=jnp.float32)
        + b2_ref[...], 0.0)
    ms_ref[...] = (jnp.dot(h, w3_ref[...], preferred_element_type=jnp.float32)
                   + b3_ref[...])


def _mention(gs, ge, qs, qe, w1, b1, w2, b2, w3, b3):
    n = gs.shape[0]
    row_spec = pl.BlockSpec((_ROWS, E), lambda i: (i, 0))
    return pl.pallas_call(
        _mention_body,
        grid=(n // _ROWS,),
        in_specs=[
            row_spec, row_spec, row_spec, row_spec,
            pl.BlockSpec((3 * E, H), lambda i: (0, 0)),
            pl.BlockSpec((1, H), lambda i: (0, 0)),
            pl.BlockSpec((H, H), lambda i: (0, 0)),
            pl.BlockSpec((1, H), lambda i: (0, 0)),
            pl.BlockSpec((H, 1), lambda i: (0, 0)),
            pl.BlockSpec((1, 1), lambda i: (0, 0)),
        ],
        out_specs=[
            pl.BlockSpec((_ROWS, 3 * E), lambda i: (i, 0)),
            pl.BlockSpec((_ROWS, 1), lambda i: (i, 0)),
        ],
        out_shape=[
            jax.ShapeDtypeStruct((n, 3 * E), jnp.float32),
            jax.ShapeDtypeStruct((n, 1), jnp.float32),
        ],
    )(gs, ge, qs, qe, w1, b1.reshape(1, H), w2, b2.reshape(1, H), w3,
      b3.reshape(1, 1))


def kernel(batch_embeds, span_starts, span_widths, Wa1, ba1, Wa2, ba2, Wa3,
           ba3, Ws1, bs1, Ws2, bs2, Ws3, bs3):
    starts = span_starts.astype(jnp.int32).reshape(-1)
    widths = span_widths.astype(jnp.int32).reshape(-1)

    emb_flat = batch_embeds.reshape(B * T, E)
    q = _attn_prefix(batch_embeds, Wa1, ba1, Wa2, ba2, Wa3, ba3)
    gs, ge, qs, qe = _sc_gather4(emb_flat, q.reshape(B * T, E), starts,
                                 widths)
    span_embeds, scores = _mention(gs, ge, qs, qe,
                                   Ws1, bs1, Ws2, bs2, Ws3, bs3)
    return span_embeds.reshape(B, S, 3 * E), scores.reshape(B, S, 1)


# speedup vs baseline: 1.2175x; 1.1385x over previous
import functools

import jax
import jax.numpy as jnp
from jax import lax
from jax.experimental import pallas as pl
from jax.experimental.pallas import tpu as pltpu
from jax.experimental.pallas import tpu_sc as plsc

B, T, E = 8, 2048, 256
S, WMAX = 512, 10
H = 150

_NC, _NS = 2, 16
_NW = _NC * _NS
_PW = (B * S) // _NW


def _attn_weighted_body(x_ref, st_ref, wd_ref, w1_ref, b1_ref, w2_ref,
                        b2_ref, w3_ref, b3_ref, wt_ref):
    x = x_ref[0]
    xb = x.astype(jnp.bfloat16)
    h = jnp.maximum(
        jnp.dot(xb, w1_ref[...], preferred_element_type=jnp.float32)
        + b1_ref[...], 0.0)
    h = jnp.maximum(
        jnp.dot(h.astype(jnp.bfloat16), w2_ref[...],
                preferred_element_type=jnp.float32) + b2_ref[...], 0.0)
    a = (jnp.dot(h.astype(jnp.bfloat16), w3_ref[...],
                 preferred_element_type=jnp.float32) + b3_ref[...])
    zb = (x * a).astype(jnp.bfloat16)

    tok = lax.broadcasted_iota(jnp.int32, (T, S), 0)
    s = st_ref[0]
    e = s + wd_ref[0]
    dt = ((tok >= s) & (tok <= e)).astype(jnp.bfloat16)
    wt_ref[0] = lax.dot_general(
        dt, zb, dimension_numbers=(((0,), (0,)), ((), ())),
        preferred_element_type=jnp.float32)


def _attn_weighted(batch_embeds, starts, widths, w1, b1, w2, b2, w3, b3):
    return pl.pallas_call(
        _attn_weighted_body,
        grid=(B,),
        in_specs=[
            pl.BlockSpec((1, T, E), lambda b: (b, 0, 0)),
            pl.BlockSpec((1, 1, S), lambda b: (b, 0, 0)),
            pl.BlockSpec((1, 1, S), lambda b: (b, 0, 0)),
            pl.BlockSpec((E, H), lambda b: (0, 0)),
            pl.BlockSpec((1, H), lambda b: (0, 0)),
            pl.BlockSpec((H, H), lambda b: (0, 0)),
            pl.BlockSpec((1, H), lambda b: (0, 0)),
            pl.BlockSpec((H, 1), lambda b: (0, 0)),
            pl.BlockSpec((1, 1), lambda b: (0, 0)),
        ],
        out_specs=pl.BlockSpec((1, S, E), lambda b: (b, 0, 0)),
        out_shape=jax.ShapeDtypeStruct((B, S, E), jnp.float32),
    )(batch_embeds, starts.reshape(B, 1, S), widths.reshape(B, 1, S),
      w1.astype(jnp.bfloat16),
      b1.reshape(1, H), w2.astype(jnp.bfloat16), b2.reshape(1, H),
      w3.astype(jnp.bfloat16), b3.reshape(1, 1))


def _sc_gather_se(table, starts, widths):
    d = table.shape[1]
    n = B * S
    mesh = plsc.VectorSubcoreMesh(core_axis_name="c", subcore_axis_name="s")
    row = jax.ShapeDtypeStruct((n, d), table.dtype)

    @functools.partial(
        pl.kernel,
        mesh=mesh,
        out_type=[row, row],
        scratch_types=[
            pltpu.VMEM((_PW,), jnp.int32),
            pltpu.VMEM((_PW,), jnp.int32),
            pltpu.VMEM((_PW,), jnp.int32),
            pltpu.VMEM((_PW,), jnp.int32),
            pltpu.VMEM((_PW, d), jnp.float32),
            pltpu.VMEM((_PW, d), jnp.float32),
            pltpu.SemaphoreType.DMA,
            pltpu.SemaphoreType.DMA,
        ],
    )
    def k(t_hbm, st_hbm, wd_hbm, oa_hbm, ob_hbm, st_v, wd_v, fa_v, fb_v,
          ra_v, rb_v, sa, sb):
        wid = lax.axis_index("s") * _NC + lax.axis_index("c")
        base = wid * _PW
        boff = (wid // (S // _PW)) * T
        pltpu.sync_copy(st_hbm.at[pl.ds(base, _PW)], st_v)
        pltpu.sync_copy(wd_hbm.at[pl.ds(base, _PW)], wd_v)

        @pl.loop(0, _PW, step=16)
        def _(i):
            s16 = st_v[pl.ds(i, 16)] + boff
            fa_v[pl.ds(i, 16)] = s16
            fb_v[pl.ds(i, 16)] = s16 + wd_v[pl.ds(i, 16)]

        ca = pltpu.async_copy(t_hbm.at[fa_v], ra_v, sa)
        cb = pltpu.async_copy(t_hbm.at[fb_v], rb_v, sb)
        ca.wait()
        pltpu.sync_copy(ra_v, oa_hbm.at[pl.ds(base, _PW)])
        cb.wait()
        pltpu.sync_copy(rb_v, ob_hbm.at[pl.ds(base, _PW)])

    return k(table, starts, widths)


_ROWS = 512


def _mention_body(gs_ref, ge_ref, wt_ref, w1_ref, b1_ref, w2_ref, b2_ref,
                  w3_ref, b3_ref, se_ref, ms_ref):
    se = jnp.concatenate([gs_ref[...], ge_ref[...], wt_ref[...]], axis=1)
    se_ref[...] = se
    h = jnp.maximum(
        jnp.dot(se.astype(jnp.bfloat16), w1_ref[...],
                preferred_element_type=jnp.float32) + b1_ref[...], 0.0)
    h = jnp.maximum(
        jnp.dot(h.astype(jnp.bfloat16), w2_ref[...],
                preferred_element_type=jnp.float32) + b2_ref[...], 0.0)
    ms_ref[...] = (jnp.dot(h.astype(jnp.bfloat16), w3_ref[...],
                           preferred_element_type=jnp.float32) + b3_ref[...])


def _mention(gs, ge, wt, w1, b1, w2, b2, w3, b3):
    n = gs.shape[0]
    row_spec = pl.BlockSpec((_ROWS, E), lambda i: (i, 0))
    return pl.pallas_call(
        _mention_body,
        grid=(n // _ROWS,),
        in_specs=[
            row_spec, row_spec, row_spec,
            pl.BlockSpec((3 * E, H), lambda i: (0, 0)),
            pl.BlockSpec((1, H), lambda i: (0, 0)),
            pl.BlockSpec((H, H), lambda i: (0, 0)),
            pl.BlockSpec((1, H), lambda i: (0, 0)),
            pl.BlockSpec((H, 1), lambda i: (0, 0)),
            pl.BlockSpec((1, 1), lambda i: (0, 0)),
        ],
        out_specs=[
            pl.BlockSpec((_ROWS, 3 * E), lambda i: (i, 0)),
            pl.BlockSpec((_ROWS, 1), lambda i: (i, 0)),
        ],
        out_shape=[
            jax.ShapeDtypeStruct((n, 3 * E), jnp.float32),
            jax.ShapeDtypeStruct((n, 1), jnp.float32),
        ],
    )(gs, ge, wt, w1.astype(jnp.bfloat16), b1.reshape(1, H),
      w2.astype(jnp.bfloat16), b2.reshape(1, H), w3.astype(jnp.bfloat16),
      b3.reshape(1, 1))


def kernel(batch_embeds, span_starts, span_widths, Wa1, ba1, Wa2, ba2, Wa3,
           ba3, Ws1, bs1, Ws2, bs2, Ws3, bs3):
    starts = span_starts.astype(jnp.int32)
    widths = span_widths.astype(jnp.int32)

    emb_flat = batch_embeds.reshape(B * T, E)
    gs, ge = _sc_gather_se(emb_flat, starts.reshape(-1), widths.reshape(-1))
    wt = _attn_weighted(batch_embeds, starts, widths, Wa1, ba1, Wa2, ba2,
                        Wa3, ba3)
    span_embeds, scores = _mention(gs, ge, wt.reshape(B * S, E), Ws1, bs1,
                                   Ws2, bs2, Ws3, bs3)
    return span_embeds.reshape(B, S, 3 * E), scores.reshape(B, S, 1)


# speedup vs baseline: 1.3013x; 1.0688x over previous
import functools

import jax
import jax.numpy as jnp
from jax import lax
from jax.experimental import pallas as pl
from jax.experimental.pallas import tpu as pltpu
from jax.experimental.pallas import tpu_sc as plsc

B, T, E = 8, 2048, 256
S, WMAX = 512, 10
H = 150

_NC, _NS = 2, 16
_NW = _NC * _NS
_PW = (B * S) // _NW
_WPB = S // _PW


def _attn_weighted_body(x_ref, st_ref, wd_ref, w1_ref, b1_ref, w2_ref,
                        b2_ref, w3_ref, b3_ref, wt_ref):
    b = pl.program_id(0)
    x = x_ref[0]
    h = jnp.maximum(
        jnp.dot(x.astype(jnp.bfloat16), w1_ref[...].astype(jnp.bfloat16),
                preferred_element_type=jnp.float32) + b1_ref[...][None, :],
        0.0)
    h = jnp.maximum(
        jnp.dot(h.astype(jnp.bfloat16), w2_ref[...].astype(jnp.bfloat16),
                preferred_element_type=jnp.float32) + b2_ref[...][None, :],
        0.0)
    a = (jnp.dot(h.astype(jnp.bfloat16), w3_ref[...].astype(jnp.bfloat16),
                 preferred_element_type=jnp.float32)
         + b3_ref[...][None, :])
    zb = (x * a).astype(jnp.bfloat16)

    tok = lax.broadcasted_iota(jnp.int32, (T, S), 0)
    s = st_ref[pl.ds(b, 1), :]
    e = s + wd_ref[pl.ds(b, 1), :]
    dt = ((tok >= s) & (tok <= e)).astype(jnp.bfloat16)
    wt_ref[0] = lax.dot_general(
        dt, zb, dimension_numbers=(((0,), (0,)), ((), ())),
        preferred_element_type=jnp.float32)


def _attn_weighted(batch_embeds, starts, widths, w1, b1, w2, b2, w3, b3):
    return pl.pallas_call(
        _attn_weighted_body,
        grid=(B,),
        in_specs=[
            pl.BlockSpec((1, T, E), lambda b: (b, 0, 0)),
            pl.BlockSpec((B, S), lambda b: (0, 0)),
            pl.BlockSpec((B, S), lambda b: (0, 0)),
            pl.BlockSpec((E, H), lambda b: (0, 0)),
            pl.BlockSpec((H,), lambda b: (0,)),
            pl.BlockSpec((H, H), lambda b: (0, 0)),
            pl.BlockSpec((H,), lambda b: (0,)),
            pl.BlockSpec((H, 1), lambda b: (0, 0)),
            pl.BlockSpec((1,), lambda b: (0,)),
        ],
        out_specs=pl.BlockSpec((1, S, E), lambda b: (b, 0, 0)),
        out_shape=jax.ShapeDtypeStruct((B, S, E), jnp.float32),
    )(batch_embeds, starts, widths, w1, b1, w2, b2, w3, b3)


def _sc_gather_se(table, starts, widths):
    d = table.shape[1]
    n = B * S
    mesh = plsc.VectorSubcoreMesh(core_axis_name="c", subcore_axis_name="s")
    row = jax.ShapeDtypeStruct((n, d), table.dtype)

    @functools.partial(
        pl.kernel,
        mesh=mesh,
        out_type=[row, row],
        scratch_types=[
            pltpu.VMEM((_PW,), jnp.int32),
            pltpu.VMEM((_PW,), jnp.int32),
            pltpu.VMEM((_PW,), jnp.int32),
            pltpu.VMEM((_PW,), jnp.int32),
            pltpu.VMEM((_PW, d), jnp.float32),
            pltpu.VMEM((_PW, d), jnp.float32),
            pltpu.SemaphoreType.DMA,
            pltpu.SemaphoreType.DMA,
        ],
    )
    def k(t_hbm, st_hbm, wd_hbm, oa_hbm, ob_hbm, st_v, wd_v, fa_v, fb_v,
          ra_v, rb_v, sa, sb):
        wid = lax.axis_index("s") * _NC + lax.axis_index("c")
        base = wid * _PW
        b = wid // _WPB
        col = (wid - b * _WPB) * _PW
        boff = b * T
        pltpu.sync_copy(st_hbm.at[b, pl.ds(col, _PW)], st_v)
        pltpu.sync_copy(wd_hbm.at[b, pl.ds(col, _PW)], wd_v)

        @pl.loop(0, _PW, step=16)
        def _(i):
            s16 = st_v[pl.ds(i, 16)] + boff
            fa_v[pl.ds(i, 16)] = s16
            fb_v[pl.ds(i, 16)] = s16 + wd_v[pl.ds(i, 16)]

        ca = pltpu.async_copy(t_hbm.at[fa_v], ra_v, sa)
        cb = pltpu.async_copy(t_hbm.at[fb_v], rb_v, sb)
        ca.wait()
        pltpu.sync_copy(ra_v, oa_hbm.at[pl.ds(base, _PW)])
        cb.wait()
        pltpu.sync_copy(rb_v, ob_hbm.at[pl.ds(base, _PW)])

    return k(table, starts, widths)


def _mention_body(gs_ref, ge_ref, wt_ref, w1_ref, b1_ref, w2_ref, b2_ref,
                  w3_ref, b3_ref, se_ref, ms_ref):
    se = jnp.concatenate([gs_ref[...], ge_ref[...], wt_ref[0]], axis=1)
    se_ref[0] = se
    h = jnp.maximum(
        jnp.dot(se.astype(jnp.bfloat16), w1_ref[...].astype(jnp.bfloat16),
                preferred_element_type=jnp.float32) + b1_ref[...][None, :],
        0.0)
    h = jnp.maximum(
        jnp.dot(h.astype(jnp.bfloat16), w2_ref[...].astype(jnp.bfloat16),
                preferred_element_type=jnp.float32) + b2_ref[...][None, :],
        0.0)
    ms_ref[0] = (jnp.dot(h.astype(jnp.bfloat16),
                         w3_ref[...].astype(jnp.bfloat16),
                         preferred_element_type=jnp.float32)
                 + b3_ref[...][None, :])


def _mention(gs, ge, wt, w1, b1, w2, b2, w3, b3):
    row_spec = pl.BlockSpec((S, E), lambda i: (i, 0))
    return pl.pallas_call(
        _mention_body,
        grid=(B,),
        in_specs=[
            row_spec, row_spec,
            pl.BlockSpec((1, S, E), lambda i: (i, 0, 0)),
            pl.BlockSpec((3 * E, H), lambda i: (0, 0)),
            pl.BlockSpec((H,), lambda i: (0,)),
            pl.BlockSpec((H, H), lambda i: (0, 0)),
            pl.BlockSpec((H,), lambda i: (0,)),
            pl.BlockSpec((H, 1), lambda i: (0, 0)),
            pl.BlockSpec((1,), lambda i: (0,)),
        ],
        out_specs=[
            pl.BlockSpec((1, S, 3 * E), lambda i: (i, 0, 0)),
            pl.BlockSpec((1, S, 1), lambda i: (i, 0, 0)),
        ],
        out_shape=[
            jax.ShapeDtypeStruct((B, S, 3 * E), jnp.float32),
            jax.ShapeDtypeStruct((B, S, 1), jnp.float32),
        ],
    )(gs, ge, wt, w1, b1, w2, b2, w3, b3)


def kernel(batch_embeds, span_starts, span_widths, Wa1, ba1, Wa2, ba2, Wa3,
           ba3, Ws1, bs1, Ws2, bs2, Ws3, bs3):
    starts = span_starts.astype(jnp.int32)
    widths = span_widths.astype(jnp.int32)

    emb_flat = batch_embeds.reshape(B * T, E)
    gs, ge = _sc_gather_se(emb_flat, starts, widths)
    wt = _attn_weighted(batch_embeds, starts, widths, Wa1, ba1, Wa2, ba2,
                        Wa3, ba3)
    span_embeds, scores = _mention(gs, ge, wt, Ws1, bs1, Ws2, bs2, Ws3, bs3)
    return span_embeds, scores
